# baseline (speedup 1.0000x reference)
import functools

import jax
import jax.numpy as jnp
from jax.experimental import pallas as pl
from jax.experimental.pallas import tpu as pltpu


def _pool_kernel(x_ref, mean_ref, std_ref, *, t_total):
    x = x_ref[...].astype(jnp.float32)
    s = jnp.sum(x, axis=-1, keepdims=True)
    ss = jnp.sum(x * x, axis=-1, keepdims=True)
    mean = s * (1.0 / jnp.float32(t_total))
    var = (ss - s * mean) * (1.0 / jnp.float32(t_total - 1))
    var = jnp.maximum(var, 0.0)
    mean_ref[...] = mean.astype(mean_ref.dtype)
    std_ref[...] = jnp.sqrt(var).astype(std_ref.dtype)


def kernel(x):
    B, C, T = x.shape
    rows = B * C
    x2 = x.reshape(rows, T)

    grid = 64
    while rows % grid or (rows // grid) % 8:
        grid //= 2
    r = rows // grid

    mean, std = pl.pallas_call(
        functools.partial(_pool_kernel, t_total=T),
        out_shape=(
            jax.ShapeDtypeStruct((rows, 1), x.dtype),
            jax.ShapeDtypeStruct((rows, 1), x.dtype),
        ),
        grid=(grid,),
        in_specs=[pl.BlockSpec((r, T), lambda i: (i, 0))],
        out_specs=[
            pl.BlockSpec((r, 1), lambda i: (i, 0)),
            pl.BlockSpec((r, 1), lambda i: (i, 0)),
        ],
        compiler_params=pltpu.CompilerParams(
            dimension_semantics=("parallel",),
        ),
    )(x2)
    return jnp.concatenate([mean.reshape(B, C), std.reshape(B, C)], axis=-1)


# speedup vs baseline: 1.5129x; 1.5129x over previous
import functools

import jax
import jax.numpy as jnp
from jax.experimental import pallas as pl
from jax.experimental.pallas import tpu as pltpu


def _pool_kernel(x_ref, mean_ref, std_ref, *, t_total):
    x = x_ref[...].astype(jnp.float32)
    s = jnp.sum(x, axis=-1)
    ss = jnp.sum(x * x, axis=-1)
    mean = s * (1.0 / jnp.float32(t_total))
    var = (ss - s * mean) * (1.0 / jnp.float32(t_total - 1))
    var = jnp.maximum(var, 0.0)
    mean_ref[...] = mean.astype(mean_ref.dtype)
    std_ref[...] = jnp.sqrt(var).astype(std_ref.dtype)


def kernel(x):
    B, C, T = x.shape
    tb = 8
    cc = C // 2

    mean, std = pl.pallas_call(
        functools.partial(_pool_kernel, t_total=T),
        out_shape=(
            jax.ShapeDtypeStruct((B, C), x.dtype),
            jax.ShapeDtypeStruct((B, C), x.dtype),
        ),
        grid=(B // tb, C // cc),
        in_specs=[pl.BlockSpec((tb, cc, T), lambda b, j: (b, j, 0))],
        out_specs=[
            pl.BlockSpec((tb, cc), lambda b, j: (b, j)),
            pl.BlockSpec((tb, cc), lambda b, j: (b, j)),
        ],
        compiler_params=pltpu.CompilerParams(
            dimension_semantics=("parallel", "parallel"),
        ),
    )(x)
    return jnp.concatenate([mean, std], axis=-1)
